# baseline (speedup 1.0000x reference)
import jax
import jax.numpy as jnp
from jax import lax
from jax.experimental import pallas as pl
from jax.experimental.pallas import tpu as pltpu
from jax.experimental.pallas import tpu_sc as plsc

_NC = 2
_NS = 16
_NW = _NC * _NS

_A_PAD = 10240
_APW = _A_PAD // _NW
_CA = 4
_CB = 40
_MB = 16
_R = 3200

_H = 128
_NBR = 32
_LG = _H // 16


def _k1_body(x1_ref, x2_ref, w1_ref, w2_ref, inp_ref):
    acc = lax.dot_general(
        x1_ref[...].astype(jnp.bfloat16), w1_ref[...], (((1,), (0,)), ((), ())),
        preferred_element_type=jnp.float32)
    acc = acc + lax.dot_general(
        x2_ref[...].astype(jnp.bfloat16), w2_ref[...], (((1,), (0,)), ((), ())),
        preferred_element_type=jnp.float32)
    inp_ref[...] = acc


def _mm_in(x1, x2, w1_bf, w2_bf):
    nb = x1.shape[0]
    fd2 = x2.shape[1]
    return pl.pallas_call(
        _k1_body,
        grid=(nb // _R,),
        in_specs=[pl.BlockSpec((_R, _H), lambda i: (i, 0)),
                  pl.BlockSpec((_R, fd2), lambda i: (i, 0)),
                  pl.BlockSpec((_H, _H), lambda i: (0, 0)),
                  pl.BlockSpec((fd2, _H), lambda i: (0, 0))],
        out_specs=pl.BlockSpec((_R, _H), lambda i: (i, 0)),
        out_shape=jax.ShapeDtypeStruct((nb, _H), jnp.float32),
    )(x1, x2, w1_bf, w2_bf)


def _unpack_bf16(p):
    u = lax.bitcast_convert_type(p, jnp.uint32)
    hb = lax.bitcast_convert_type((u >> 16).astype(jnp.uint16), jnp.bfloat16)
    lb = lax.bitcast_convert_type((u & jnp.uint32(0xFFFF)).astype(jnp.uint16),
                                  jnp.bfloat16)
    return jnp.concatenate([hb, lb], axis=1)


def _k3_body(t_ref, inp_ref, w_ref, out_ref):
    acc = lax.dot_general(_unpack_bf16(t_ref[...]), w_ref[...],
                          (((1,), (0,)), ((), ())),
                          preferred_element_type=jnp.float32)
    out_ref[...] = acc + inp_ref[...]


def _mm_update(t, inp, w_h_t_bf):
    nb = t.shape[0]
    return pl.pallas_call(
        _k3_body,
        grid=(nb // _R,),
        in_specs=[pl.BlockSpec((_R, _H // 2), lambda i: (i, 0)),
                  pl.BlockSpec((_R, _H), lambda i: (i, 0)),
                  pl.BlockSpec((_H, _H), lambda i: (0, 0))],
        out_specs=pl.BlockSpec((_R, _H), lambda i: (i, 0)),
        out_shape=jax.ShapeDtypeStruct((nb, _H), jnp.float32),
    )(t, inp, w_h_t_bf)


def _k4_body(fa_ref, am_ref, w1_ref, w2_ref, b_ref, inv_ref, out_ref):
    h = lax.dot_general(fa_ref[...], w1_ref[...], (((1,), (0,)), ((), ())),
                        preferred_element_type=jnp.float32)
    h = h + lax.dot_general(am_ref[...], w2_ref[...], (((1,), (0,)), ((), ())),
                            preferred_element_type=jnp.float32)
    h = jnp.maximum(h + b_ref[...], 0.0)
    row = lax.broadcasted_iota(jnp.int32, (_MB, _MB * 20), 0)
    col = lax.broadcasted_iota(jnp.int32, (_MB, _MB * 20), 1)
    sel = (col >= row * 20) & (col < (row + 1) * 20)
    p = jnp.where(sel, 1.0, 0.0)
    mol = lax.dot_general(p, h, (((1,), (0,)), ((), ())),
                          preferred_element_type=jnp.float32)
    out_ref[...] = mol * inv_ref[...]


def _mol_out(fa_pad, a_msg_pad, w1_t, w2_t, bias, inv_pad):
    n_mol_pad = inv_pad.shape[0]
    apb = _MB * 20
    return pl.pallas_call(
        _k4_body,
        grid=(n_mol_pad // _MB,),
        in_specs=[pl.BlockSpec((apb, _H), lambda i: (i, 0)),
                  pl.BlockSpec((apb, _H), lambda i: (i, 0)),
                  pl.BlockSpec((_H, _H), lambda i: (0, 0)),
                  pl.BlockSpec((_H, _H), lambda i: (0, 0)),
                  pl.BlockSpec((1, _H), lambda i: (0, 0)),
                  pl.BlockSpec((_MB, _H), lambda i: (i, 0))],
        out_specs=pl.BlockSpec((_MB, _H), lambda i: (i, 0)),
        out_shape=jax.ShapeDtypeStruct((n_mol_pad, _H), jnp.float32),
    )(fa_pad, a_msg_pad, w1_t, w2_t, bias, inv_pad)


_SC_MESH = plsc.VectorSubcoreMesh(core_axis_name="c", subcore_axis_name="s")


def _worker_id():
    return lax.axis_index("s") * _NC + lax.axis_index("c")


def _s1_body(pre_hbm, a2b_hbm, out_hbm, idx_v, rows0, rows1, acc0, acc1,
             gs0, gs1, ss0, ss1):
    wid = _worker_id()
    base = wid * _APW
    nch = _APW // _CA
    rows = (rows0, rows1)
    accs = (acc0, acc1)
    gsems = (gs0, gs1)
    ssems = (ss0, ss1)

    pltpu.sync_copy(a2b_hbm.at[pl.ds(base, _APW)], idx_v)

    def start(g, b):
        for a in range(_CA):
            pltpu.async_copy(pre_hbm.at[idx_v.at[g * _CA + a]],
                             rows[b].at[pl.ds(a * _NBR, _NBR)], gsems[b])

    def wait_gather(b):
        for a in range(_CA):
            pltpu.make_async_copy(pre_hbm.at[idx_v.at[0]],
                                  rows[b].at[pl.ds(a * _NBR, _NBR)],
                                  gsems[b]).wait()

    def wait_store(b):
        pltpu.make_async_copy(accs[b], out_hbm.at[pl.ds(base, _CA)],
                              ssems[b]).wait()

    start(0, 0)
    start(1, 1)

    def pair(i, carry):
        for b in range(2):
            g = 2 * i + b
            wait_gather(b)

            @pl.when(i >= 1)
            def _():
                wait_store(b)

            @plsc.parallel_loop(0, _CA * _LG, 1, unroll=2)
            def _(k):
                a = lax.shift_right_logical(k, 3)
                lane = jnp.bitwise_and(k, _LG - 1) * 16
                row0 = a * _NBR
                vals = [jnp.maximum(rows[b][row0 + r, pl.ds(lane, 16)], 0.0)
                        for r in range(_NBR)]
                while len(vals) > 1:
                    vals = [vals[m] + vals[m + 1]
                            for m in range(0, len(vals), 2)]
                accs[b][a, pl.ds(lane, 16)] = vals[0]
            dst = pl.multiple_of(base + g * _CA, _CA)
            pltpu.async_copy(accs[b], out_hbm.at[pl.ds(dst, _CA)], ssems[b])

            @pl.when(g + 2 < nch)
            def _():
                start(g + 2, b)
        return carry

    lax.fori_loop(0, nch // 2, pair, 0)
    wait_store(0)
    wait_store(1)


def _gather_sum(pre, a2b_pad):
    return pl.kernel(
        _s1_body,
        out_type=jax.ShapeDtypeStruct((_A_PAD, _H), jnp.float32),
        mesh=_SC_MESH,
        scratch_types=[
            pltpu.VMEM((_APW, _NBR), jnp.int32),
            pltpu.VMEM((_CA * _NBR, _H), jnp.float32),
            pltpu.VMEM((_CA * _NBR, _H), jnp.float32),
            pltpu.VMEM((_CA, _H), jnp.float32),
            pltpu.VMEM((_CA, _H), jnp.float32),
            pltpu.SemaphoreType.DMA,
            pltpu.SemaphoreType.DMA,
            pltpu.SemaphoreType.DMA,
            pltpu.SemaphoreType.DMA,
        ],
    )(pre, a2b_pad)


def _s2_body(amsg_hbm, pre_hbm, b2a_hbm, b2revb_hbm, out_hbm,
             i1_v, i2_v, g1_0, g1_1, g2_0, g2_1, t0, t1,
             gs0, gs1, ss0, ss1):
    wid = _worker_id()
    n_bonds = out_hbm.shape[0]
    bpw = n_bonds // _NW
    base = wid * bpw
    nch = bpw // _CB
    g1s = (g1_0, g1_1)
    g2s = (g2_0, g2_1)
    ts = (t0, t1)
    gsems = (gs0, gs1)
    ssems = (ss0, ss1)

    pltpu.sync_copy(b2a_hbm.at[pl.ds(base, bpw)], i1_v)
    pltpu.sync_copy(b2revb_hbm.at[pl.ds(base, bpw)], i2_v)

    def start(g, b):
        off = pl.multiple_of(g * _CB, _CB)
        pltpu.async_copy(amsg_hbm.at[i1_v.at[pl.ds(off, _CB)]],
                         g1s[b], gsems[b])
        pltpu.async_copy(pre_hbm.at[i2_v.at[pl.ds(off, _CB)]],
                         g2s[b], gsems[b])

    def wait_gather(b):
        pltpu.make_async_copy(amsg_hbm.at[i1_v.at[pl.ds(0, _CB)]],
                              g1s[b], gsems[b]).wait()
        pltpu.make_async_copy(pre_hbm.at[i2_v.at[pl.ds(0, _CB)]],
                              g2s[b], gsems[b]).wait()

    def wait_store(b):
        pltpu.make_async_copy(ts[b], out_hbm.at[pl.ds(base, _CB)],
                              ssems[b]).wait()

    start(0, 0)
    start(1, 1)

    def pair(i, carry):
        for b in range(2):
            g = 2 * i + b
            wait_gather(b)

            @pl.when(i >= 1)
            def _():
                wait_store(b)

            @plsc.parallel_loop(0, _CB * (_LG // 2), 1, unroll=4)
            def _(k):
                r = lax.shift_right_logical(k, 2)
                lane = jnp.bitwise_and(k, _LG // 2 - 1) * 16
                hi = (g1s[b][r, pl.ds(lane, 16)]
                      - jnp.maximum(g2s[b][r, pl.ds(lane, 16)], 0.0))
                lo = (g1s[b][r, pl.ds(lane + 64, 16)]
                      - jnp.maximum(g2s[b][r, pl.ds(lane + 64, 16)], 0.0))
                uh = lax.bitcast_convert_type(hi, jnp.int32) + jnp.int32(0x8000)
                ul = lax.bitcast_convert_type(lo, jnp.int32) + jnp.int32(0x8000)
                ts[b][r, pl.ds(lane, 16)] = jnp.bitwise_or(
                    jnp.bitwise_and(uh, jnp.int32(-65536)),
                    lax.shift_right_logical(ul, 16))
            dst = pl.multiple_of(base + g * _CB, _CB)
            pltpu.async_copy(ts[b], out_hbm.at[pl.ds(dst, _CB)], ssems[b])

            @pl.when(g + 2 < nch)
            def _():
                start(g + 2, b)
        return carry

    lax.fori_loop(0, nch // 2, pair, 0)
    wait_store(0)
    wait_store(1)


def _gather_diff(a_msg_pad, pre, b2a, b2revb):
    nb = pre.shape[0]
    return pl.kernel(
        _s2_body,
        out_type=jax.ShapeDtypeStruct((nb, _H // 2), jnp.int32),
        mesh=_SC_MESH,
        scratch_types=[
            pltpu.VMEM((nb // _NW,), jnp.int32),
            pltpu.VMEM((nb // _NW,), jnp.int32),
            pltpu.VMEM((_CB, _H), jnp.float32),
            pltpu.VMEM((_CB, _H), jnp.float32),
            pltpu.VMEM((_CB, _H), jnp.float32),
            pltpu.VMEM((_CB, _H), jnp.float32),
            pltpu.VMEM((_CB, _H // 2), jnp.int32),
            pltpu.VMEM((_CB, _H // 2), jnp.int32),
            pltpu.SemaphoreType.DMA,
            pltpu.SemaphoreType.DMA,
            pltpu.SemaphoreType.DMA,
            pltpu.SemaphoreType.DMA,
        ],
    )(a_msg_pad, pre, b2a, b2revb)


def kernel(f_atoms, f_bonds, a2b, b2a, b2revb, a_scope, W_i, W_h, W_o_w, W_o_b):
    n_atoms, atom_fdim = f_atoms.shape
    n_mols = a_scope.shape[0]
    n_mol_pad = _A_PAD // 20

    n_bonds = f_bonds.shape[0]
    max_nb = a2b.shape[1]
    pad_idx = (jnp.arange((_A_PAD - n_atoms) * max_nb, dtype=jnp.int32)
               % n_bonds).reshape(_A_PAD - n_atoms, max_nb)
    a2b_pad = jnp.concatenate([a2b.astype(jnp.int32), pad_idx], axis=0)
    fa_pad = jnp.concatenate(
        [f_atoms, jnp.zeros((_A_PAD - n_atoms, atom_fdim), jnp.float32)])
    inv = jnp.broadcast_to(
        (1.0 / a_scope[:, 1].astype(jnp.float32))[:, None], (n_mols, _H))
    inv_pad = jnp.concatenate(
        [inv, jnp.ones((n_mol_pad - n_mols, _H), jnp.float32)])
    w_i1_bf = W_i[:, :_H].T.astype(jnp.bfloat16)
    w_i2_bf = W_i[:, _H:].T.astype(jnp.bfloat16)
    w_h_t = W_h.T.astype(jnp.bfloat16)
    w1_t = W_o_w[:, :atom_fdim].T
    w2_t = W_o_w[:, atom_fdim:].T
    bias = W_o_b[None, :]
    b2a32 = b2a.astype(jnp.int32)
    b2revb32 = b2revb.astype(jnp.int32)

    x1 = f_bonds[:, :_H]
    x2 = f_bonds[:, _H:]
    inp = _mm_in(x1, x2, w_i1_bf, w_i2_bf)
    pre = inp
    for _ in range(2):
        a_msg = _gather_sum(pre, a2b_pad)
        t = _gather_diff(a_msg, pre, b2a32, b2revb32)
        pre = _mm_update(t, inp, w_h_t)
    a_msg = _gather_sum(pre, a2b_pad)
    mol = _mol_out(fa_pad, a_msg, w1_t, w2_t, bias, inv_pad)
    return mol[:n_mols]


# speedup vs baseline: 1.1528x; 1.1528x over previous
import jax
import jax.numpy as jnp
from jax import lax
from jax.experimental import pallas as pl
from jax.experimental.pallas import tpu as pltpu
from jax.experimental.pallas import tpu_sc as plsc

_NC = 2
_NS = 16
_NW = _NC * _NS

_A_PAD = 10240
_APW = _A_PAD // _NW
_CA = 4
_CB = 40
_MB = 16
_R = 3200

_H = 128
_NBR = 32
_LG = _H // 16


def _k1_body(x_ref, w_ref, inp_ref):
    inp_ref[...] = lax.dot_general(
        x_ref[...].astype(jnp.bfloat16), w_ref[...], (((1,), (0,)), ((), ())),
        preferred_element_type=jnp.float32)


def _mm_in(f_bonds, w_i_t):
    nb, fd = f_bonds.shape
    return pl.pallas_call(
        _k1_body,
        grid=(nb // _R,),
        in_specs=[pl.BlockSpec((_R, fd), lambda i: (i, 0)),
                  pl.BlockSpec((fd, _H), lambda i: (0, 0))],
        out_specs=pl.BlockSpec((_R, _H), lambda i: (i, 0)),
        out_shape=jax.ShapeDtypeStruct((nb, _H), jnp.float32),
    )(f_bonds, w_i_t)


def _unpack_bf16(p):
    u = lax.bitcast_convert_type(p, jnp.uint32)
    hb = lax.bitcast_convert_type((u >> 16).astype(jnp.uint16), jnp.bfloat16)
    lb = lax.bitcast_convert_type((u & jnp.uint32(0xFFFF)).astype(jnp.uint16),
                                  jnp.bfloat16)
    return jnp.concatenate([hb, lb], axis=1)


def _k3_body(t_ref, inp_ref, w_ref, out_ref):
    acc = lax.dot_general(_unpack_bf16(t_ref[...]), w_ref[...],
                          (((1,), (0,)), ((), ())),
                          preferred_element_type=jnp.float32)
    out_ref[...] = acc + inp_ref[...]


def _mm_update(t, inp, w_h_t_bf):
    nb = t.shape[0]
    return pl.pallas_call(
        _k3_body,
        grid=(nb // _R,),
        in_specs=[pl.BlockSpec((_R, _H // 2), lambda i: (i, 0)),
                  pl.BlockSpec((_R, _H), lambda i: (i, 0)),
                  pl.BlockSpec((_H, _H), lambda i: (0, 0))],
        out_specs=pl.BlockSpec((_R, _H), lambda i: (i, 0)),
        out_shape=jax.ShapeDtypeStruct((nb, _H), jnp.float32),
    )(t, inp, w_h_t_bf)


def _k4_body(fa_ref, am_ref, w1_ref, w2_ref, b_ref, inv_ref, out_ref):
    h = lax.dot_general(fa_ref[...], w1_ref[...], (((1,), (0,)), ((), ())),
                        preferred_element_type=jnp.float32)
    h = h + lax.dot_general(am_ref[...], w2_ref[...], (((1,), (0,)), ((), ())),
                            preferred_element_type=jnp.float32)
    h = jnp.maximum(h + b_ref[...], 0.0)
    row = lax.broadcasted_iota(jnp.int32, (_MB, _MB * 20), 0)
    col = lax.broadcasted_iota(jnp.int32, (_MB, _MB * 20), 1)
    sel = (col >= row * 20) & (col < (row + 1) * 20)
    p = jnp.where(sel, 1.0, 0.0)
    mol = lax.dot_general(p, h, (((1,), (0,)), ((), ())),
                          preferred_element_type=jnp.float32)
    out_ref[...] = mol * inv_ref[...]


def _mol_out(fa_pad, a_msg_pad, w1_t, w2_t, bias, inv_pad):
    n_mol_pad = inv_pad.shape[0]
    apb = _MB * 20
    return pl.pallas_call(
        _k4_body,
        grid=(n_mol_pad // _MB,),
        in_specs=[pl.BlockSpec((apb, _H), lambda i: (i, 0)),
                  pl.BlockSpec((apb, _H), lambda i: (i, 0)),
                  pl.BlockSpec((_H, _H), lambda i: (0, 0)),
                  pl.BlockSpec((_H, _H), lambda i: (0, 0)),
                  pl.BlockSpec((1, _H), lambda i: (0, 0)),
                  pl.BlockSpec((_MB, _H), lambda i: (i, 0))],
        out_specs=pl.BlockSpec((_MB, _H), lambda i: (i, 0)),
        out_shape=jax.ShapeDtypeStruct((n_mol_pad, _H), jnp.float32),
    )(fa_pad, a_msg_pad, w1_t, w2_t, bias, inv_pad)


_SC_MESH = plsc.VectorSubcoreMesh(core_axis_name="c", subcore_axis_name="s")


def _worker_id():
    return lax.axis_index("s") * _NC + lax.axis_index("c")


def _s1_body(pre_hbm, a2b_hbm, out_hbm, idx_v, rows0, rows1, acc0, acc1,
             gs0, gs1, ss0, ss1):
    wid = _worker_id()
    base = wid * _APW
    nch = _APW // _CA
    rows = (rows0, rows1)
    accs = (acc0, acc1)
    gsems = (gs0, gs1)
    ssems = (ss0, ss1)

    pltpu.sync_copy(a2b_hbm.at[pl.ds(base, _APW)], idx_v)

    def start(g, b):
        for a in range(_CA):
            pltpu.async_copy(pre_hbm.at[idx_v.at[g * _CA + a]],
                             rows[b].at[pl.ds(a * _NBR, _NBR)], gsems[b])

    def wait_gather(b):
        for a in range(_CA):
            pltpu.make_async_copy(pre_hbm.at[idx_v.at[0]],
                                  rows[b].at[pl.ds(a * _NBR, _NBR)],
                                  gsems[b]).wait()

    def wait_store(b):
        pltpu.make_async_copy(accs[b], out_hbm.at[pl.ds(base, _CA)],
                              ssems[b]).wait()

    start(0, 0)
    start(1, 1)

    def pair(i, carry):
        for b in range(2):
            g = 2 * i + b
            wait_gather(b)

            @pl.when(i >= 1)
            def _():
                wait_store(b)

            @plsc.parallel_loop(0, _CA * _LG, 1, unroll=2)
            def _(k):
                a = lax.shift_right_logical(k, 3)
                lane = jnp.bitwise_and(k, _LG - 1) * 16
                row0 = a * _NBR
                vals = [jnp.maximum(rows[b][row0 + r, pl.ds(lane, 16)], 0.0)
                        for r in range(_NBR)]
                while len(vals) > 1:
                    vals = [vals[m] + vals[m + 1]
                            for m in range(0, len(vals), 2)]
                accs[b][a, pl.ds(lane, 16)] = vals[0]
            dst = pl.multiple_of(base + g * _CA, _CA)
            pltpu.async_copy(accs[b], out_hbm.at[pl.ds(dst, _CA)], ssems[b])

            @pl.when(g + 2 < nch)
            def _():
                start(g + 2, b)
        return carry

    lax.fori_loop(0, nch // 2, pair, 0)
    wait_store(0)
    wait_store(1)


def _gather_sum(pre, a2b_pad):
    return pl.kernel(
        _s1_body,
        out_type=jax.ShapeDtypeStruct((_A_PAD, _H), jnp.float32),
        mesh=_SC_MESH,
        scratch_types=[
            pltpu.VMEM((_APW, _NBR), jnp.int32),
            pltpu.VMEM((_CA * _NBR, _H), jnp.float32),
            pltpu.VMEM((_CA * _NBR, _H), jnp.float32),
            pltpu.VMEM((_CA, _H), jnp.float32),
            pltpu.VMEM((_CA, _H), jnp.float32),
            pltpu.SemaphoreType.DMA,
            pltpu.SemaphoreType.DMA,
            pltpu.SemaphoreType.DMA,
            pltpu.SemaphoreType.DMA,
        ],
    )(pre, a2b_pad)


def _s2_body(amsg_hbm, pre_hbm, b2a_hbm, b2revb_hbm, out_hbm,
             i1_v, i2_v, g1_0, g1_1, g1_2, g1_3, g2_0, g2_1, g2_2, g2_3,
             t0, t1, t2, t3, gs0, gs1, gs2, gs3, ss0, ss1, ss2, ss3):
    wid = _worker_id()
    n_bonds = out_hbm.shape[0]
    bpw = n_bonds // _NW
    base = wid * bpw
    nch = bpw // _CB
    g1s = (g1_0, g1_1, g1_2, g1_3)
    g2s = (g2_0, g2_1, g2_2, g2_3)
    ts = (t0, t1, t2, t3)
    gsems = (gs0, gs1, gs2, gs3)
    ssems = (ss0, ss1, ss2, ss3)

    pltpu.sync_copy(b2a_hbm.at[pl.ds(base, bpw)], i1_v)
    pltpu.sync_copy(b2revb_hbm.at[pl.ds(base, bpw)], i2_v)

    def start(g, b):
        off = pl.multiple_of(g * _CB, _CB)
        pltpu.async_copy(amsg_hbm.at[i1_v.at[pl.ds(off, _CB)]],
                         g1s[b], gsems[b])
        pltpu.async_copy(pre_hbm.at[i2_v.at[pl.ds(off, _CB)]],
                         g2s[b], gsems[b])

    def wait_gather(b):
        pltpu.make_async_copy(amsg_hbm.at[i1_v.at[pl.ds(0, _CB)]],
                              g1s[b], gsems[b]).wait()
        pltpu.make_async_copy(pre_hbm.at[i2_v.at[pl.ds(0, _CB)]],
                              g2s[b], gsems[b]).wait()

    def wait_store(b):
        pltpu.make_async_copy(ts[b], out_hbm.at[pl.ds(base, _CB)],
                              ssems[b]).wait()

    def compute_store(g, b):
        @plsc.parallel_loop(0, _CB * (_LG // 2), 1, unroll=4)
        def _(k):
            r = lax.shift_right_logical(k, 2)
            lane = jnp.bitwise_and(k, _LG // 2 - 1) * 16
            hi = (g1s[b][r, pl.ds(lane, 16)]
                  - jnp.maximum(g2s[b][r, pl.ds(lane, 16)], 0.0))
            lo = (g1s[b][r, pl.ds(lane + 64, 16)]
                  - jnp.maximum(g2s[b][r, pl.ds(lane + 64, 16)], 0.0))
            uh = lax.bitcast_convert_type(hi, jnp.int32) + jnp.int32(0x8000)
            ul = lax.bitcast_convert_type(lo, jnp.int32) + jnp.int32(0x8000)
            ts[b][r, pl.ds(lane, 16)] = jnp.bitwise_or(
                jnp.bitwise_and(uh, jnp.int32(-65536)),
                lax.shift_right_logical(ul, 16))
        dst = pl.multiple_of(base + g * _CB, _CB)
        pltpu.async_copy(ts[b], out_hbm.at[pl.ds(dst, _CB)], ssems[b])

    for b in range(4):
        start(b, b)

    def quad(i, carry):
        for b in range(4):
            g = 4 * i + b
            wait_gather(b)

            @pl.when(i >= 1)
            def _():
                wait_store(b)

            compute_store(g, b)

            @pl.when(g + 4 < nch)
            def _():
                start(g + 4, b)
        return carry

    nquad = nch // 4
    lax.fori_loop(0, nquad, quad, 0)
    for b in range(nch - 4 * nquad):
        g = 4 * nquad + b
        wait_gather(b)
        wait_store(b)
        compute_store(g, b)
    for b in range(4):
        wait_store(b)


def _gather_diff(a_msg_pad, pre, b2a, b2revb):
    nb = pre.shape[0]
    return pl.kernel(
        _s2_body,
        out_type=jax.ShapeDtypeStruct((nb, _H // 2), jnp.int32),
        mesh=_SC_MESH,
        scratch_types=(
            [pltpu.VMEM((nb // _NW,), jnp.int32)] * 2
            + [pltpu.VMEM((_CB, _H), jnp.float32)] * 8
            + [pltpu.VMEM((_CB, _H // 2), jnp.int32)] * 4
            + [pltpu.SemaphoreType.DMA] * 8
        ),
    )(a_msg_pad, pre, b2a, b2revb)


def kernel(f_atoms, f_bonds, a2b, b2a, b2revb, a_scope, W_i, W_h, W_o_w, W_o_b):
    n_atoms, atom_fdim = f_atoms.shape
    n_mols = a_scope.shape[0]
    n_mol_pad = _A_PAD // 20

    n_bonds = f_bonds.shape[0]
    max_nb = a2b.shape[1]
    pad_idx = (jnp.arange((_A_PAD - n_atoms) * max_nb, dtype=jnp.int32)
               % n_bonds).reshape(_A_PAD - n_atoms, max_nb)
    a2b_pad = jnp.concatenate([a2b.astype(jnp.int32), pad_idx], axis=0)
    fa_pad = jnp.concatenate(
        [f_atoms, jnp.zeros((_A_PAD - n_atoms, atom_fdim), jnp.float32)])
    inv = jnp.broadcast_to(
        (1.0 / a_scope[:, 1].astype(jnp.float32))[:, None], (n_mols, _H))
    inv_pad = jnp.concatenate(
        [inv, jnp.ones((n_mol_pad - n_mols, _H), jnp.float32)])
    w_i_t = W_i.T.astype(jnp.bfloat16)
    w_h_t = W_h.T.astype(jnp.bfloat16)
    w1_t = W_o_w[:, :atom_fdim].T
    w2_t = W_o_w[:, atom_fdim:].T
    bias = W_o_b[None, :]
    b2a32 = b2a.astype(jnp.int32)
    b2revb32 = b2revb.astype(jnp.int32)

    inp = _mm_in(f_bonds, w_i_t)
    pre = inp
    for _ in range(2):
        a_msg = _gather_sum(pre, a2b_pad)
        t = _gather_diff(a_msg, pre, b2a32, b2revb32)
        pre = _mm_update(t, inp, w_h_t)
    a_msg = _gather_sum(pre, a2b_pad)
    mol = _mol_out(fa_pad, a_msg, w1_t, w2_t, bias, inv_pad)
    return mol[:n_mols]
